# baseline (speedup 1.0000x reference)
import functools

import jax
import jax.numpy as jnp
from jax import lax
from jax.experimental import pallas as pl
from jax.experimental.pallas import tpu as pltpu
from jax.experimental.pallas import tpu_sc as plsc

POP = 1000000
DELAY = 8
NWORK = 31
CHUNK = 252 * 128
HALF = CHUNK // 3
TAIL = POP - NWORK * CHUNK

_mesh = plsc.VectorSubcoreMesh(core_axis_name="c", subcore_axis_name="s")


@functools.partial(
    pl.kernel,
    mesh=_mesh,
    out_type=jax.ShapeDtypeStruct((POP,), jnp.float32),
    scratch_types=[
        pltpu.VMEM((16,), jnp.int32),
        pltpu.VMEM((1, HALF), jnp.float32),
        pltpu.VMEM((1, HALF), jnp.float32),
        pltpu.VMEM((1, HALF), jnp.float32),
        pltpu.VMEM((8, TAIL), jnp.float32),
        pltpu.SemaphoreType.DMA,
        pltpu.SemaphoreType.DMA,
        pltpu.SemaphoreType.DMA,
        pltpu.SemaphoreType.DMA,
        pltpu.SemaphoreType.DMA,
        pltpu.SemaphoreType.DMA,
    ],
)
def _delayed_row_copy(hist_hbm, out_hbm, idx_v, row_a, row_b, row_c, tail_v,
                      sem_ga, sem_gb, sem_gc, sem_oa, sem_ob, sem_oc):
    wid = lax.axis_index("s") * 2 + lax.axis_index("c")
    idx_v[...] = jnp.full((16,), DELAY - 1, jnp.int32)
    idx1 = idx_v.at[pl.ds(0, 1)]

    @pl.when(wid < NWORK)
    def _():
        base = wid * CHUNK
        ga = pltpu.async_copy(
            hist_hbm.at[idx1, pl.ds(base, HALF)], row_a, sem_ga)
        gb = pltpu.async_copy(
            hist_hbm.at[idx1, pl.ds(base + HALF, HALF)], row_b, sem_gb)
        gc = pltpu.async_copy(
            hist_hbm.at[idx1, pl.ds(base + 2 * HALF, HALF)], row_c, sem_gc)
        ga.wait()
        oa = pltpu.async_copy(
            row_a.at[0], out_hbm.at[pl.ds(base, HALF)], sem_oa)
        gb.wait()
        ob = pltpu.async_copy(
            row_b.at[0], out_hbm.at[pl.ds(base + HALF, HALF)], sem_ob)
        gc.wait()
        oc = pltpu.async_copy(
            row_c.at[0], out_hbm.at[pl.ds(base + 2 * HALF, HALF)], sem_oc)
        oa.wait()
        ob.wait()
        oc.wait()

    @pl.when(wid == NWORK)
    def _():
        base = NWORK * CHUNK
        pltpu.sync_copy(hist_hbm.at[pl.ds(0, 8), pl.ds(base, TAIL)], tail_v)
        pltpu.sync_copy(tail_v.at[DELAY - 1], out_hbm.at[pl.ds(base, TAIL)])


def kernel(s, spike_history):
    return _delayed_row_copy(spike_history)


# speedup vs baseline: 1.0052x; 1.0052x over previous
import functools

import jax
import jax.numpy as jnp
from jax import lax
from jax.experimental import pallas as pl
from jax.experimental.pallas import tpu as pltpu
from jax.experimental.pallas import tpu_sc as plsc

POP = 1000000
DELAY = 8
NWORK = 31
CHUNK = 252 * 128
HALF = CHUNK // 2
TAIL = POP - NWORK * CHUNK

_mesh = plsc.VectorSubcoreMesh(core_axis_name="c", subcore_axis_name="s")


@functools.partial(
    pl.kernel,
    mesh=_mesh,
    out_type=jax.ShapeDtypeStruct((POP,), jnp.float32),
    scratch_types=[
        pltpu.VMEM((16,), jnp.int32),
        pltpu.VMEM((1, HALF), jnp.float32),
        pltpu.VMEM((1, HALF), jnp.float32),
        pltpu.VMEM((8, TAIL), jnp.float32),
        pltpu.SemaphoreType.DMA,
        pltpu.SemaphoreType.DMA,
        pltpu.SemaphoreType.DMA,
        pltpu.SemaphoreType.DMA,
    ],
)
def _delayed_row_copy(hist_hbm, out_hbm, idx_v, row_a, row_b, tail_v,
                      sem_ga, sem_gb, sem_oa, sem_ob):
    wid = lax.axis_index("s") * 2 + lax.axis_index("c")
    idx_v[...] = jnp.full((16,), DELAY - 1, jnp.int32)
    idx1 = idx_v.at[pl.ds(0, 1)]

    @pl.when(wid < NWORK)
    def _():
        base = wid * CHUNK
        ga = pltpu.async_copy(
            hist_hbm.at[idx1, pl.ds(base, HALF)], row_a, sem_ga)
        gb = pltpu.async_copy(
            hist_hbm.at[idx1, pl.ds(base + HALF, HALF)], row_b, sem_gb)
        ga.wait()
        oa = pltpu.async_copy(
            row_a.at[0], out_hbm.at[pl.ds(base, HALF)], sem_oa)
        gb.wait()
        ob = pltpu.async_copy(
            row_b.at[0], out_hbm.at[pl.ds(base + HALF, HALF)], sem_ob)
        oa.wait()
        ob.wait()

    @pl.when(wid == NWORK)
    def _():
        base = NWORK * CHUNK
        pltpu.sync_copy(hist_hbm.at[pl.ds(0, 8), pl.ds(base, TAIL)], tail_v)
        pltpu.sync_copy(tail_v.at[DELAY - 1], out_hbm.at[pl.ds(base, TAIL)])


def kernel(s, spike_history):
    return _delayed_row_copy(spike_history)
